# baseline (speedup 1.0000x reference)
import functools

import jax
import jax.numpy as jnp
from jax import lax
from jax.experimental import pallas as pl
from jax.experimental.pallas import tpu as pltpu
from jax.experimental.pallas import tpu_sc as plsc

B, T, D = 16, 576, 256
M = B * T
K = 8192
NQ = 4
TM = 512
TK = 1024

NC, NS = 2, 16
NW = NC * NS
BPW = M // NW
IDX_CH, IDX_CS = 3, 96


def _argmax_body(r_ref, n_ref, e_ref, idx_ref, rmax, ridx):
    k = pl.program_id(0)
    m = pl.program_id(1)
    nk = pl.num_programs(0)
    rn = r_ref[...] / (n_ref[...] + 1e-12)
    s = lax.dot_general(rn, e_ref[...], (((1,), (1,)), ((), ())),
                        preferred_element_type=jnp.float32)
    lmax = jnp.max(s, axis=1, keepdims=True)
    iota = lax.broadcasted_iota(jnp.int32, s.shape, 1)
    lidx = jnp.min(jnp.where(s == lmax, iota, jnp.int32(2**30)),
                   axis=1, keepdims=True) + k * TK
    msl = pl.ds(m * TM, TM)

    @pl.when(k == 0)
    def _():
        rmax[msl] = lmax
        ridx[msl] = lidx

    @pl.when(k > 0)
    def _():
        better = lmax > rmax[msl]
        rmax[msl] = jnp.where(better, lmax, rmax[msl])
        ridx[msl] = jnp.where(better, lidx, ridx[msl])

    @pl.when(k == nk - 1)
    def _():
        idx_ref[...] = ridx[msl]


_argmax = pl.pallas_call(
    _argmax_body,
    grid=(K // TK, M // TM),
    in_specs=[
        pl.BlockSpec((TM, D), lambda k, m: (m, 0)),
        pl.BlockSpec((TM, 1), lambda k, m: (m, 0)),
        pl.BlockSpec((TK, D), lambda k, m: (k, 0)),
    ],
    out_specs=pl.BlockSpec((TM, 1), lambda k, m: (m, 0)),
    out_shape=jax.ShapeDtypeStruct((M, 1), jnp.int32),
    scratch_shapes=[
        pltpu.VMEM((M, 1), jnp.float32),
        pltpu.VMEM((M, 1), jnp.int32),
    ],
)


def _update_body(r_ref, q_ref, s_ref, rnew_ref, ss_ref):
    m = pl.program_id(0)
    rn = r_ref[...] - s_ref[...] * q_ref[...]
    rnew_ref[...] = rn

    @pl.when(m == 0)
    def _():
        ss_ref[...] = jnp.zeros((1, 1), jnp.float32)

    ss_ref[...] += jnp.sum(rn * rn).reshape(1, 1)


_update = pl.pallas_call(
    _update_body,
    grid=(M // TM,),
    in_specs=[
        pl.BlockSpec((TM, D), lambda m: (m, 0)),
        pl.BlockSpec((TM, D), lambda m: (m, 0)),
        pl.BlockSpec((TM, 1), lambda m: (m, 0)),
    ],
    out_specs=[
        pl.BlockSpec((TM, D), lambda m: (m, 0)),
        pl.BlockSpec((1, 1), lambda m: (0, 0)),
    ],
    out_shape=[
        jax.ShapeDtypeStruct((M, D), jnp.float32),
        jax.ShapeDtypeStruct((1, 1), jnp.float32),
    ],
)


def _update_final_body(r_ref, q_ref, s_ref, x_ref, rnew_ref, ss_ref, xq_ref):
    m = pl.program_id(0)
    rn = r_ref[...] - s_ref[...] * q_ref[...]
    rnew_ref[...] = rn
    xq_ref[...] = x_ref[...] - rn

    @pl.when(m == 0)
    def _():
        ss_ref[...] = jnp.zeros((1, 1), jnp.float32)

    ss_ref[...] += jnp.sum(rn * rn).reshape(1, 1)


_update_final = pl.pallas_call(
    _update_final_body,
    grid=(M // TM,),
    in_specs=[
        pl.BlockSpec((TM, D), lambda m: (m, 0)),
        pl.BlockSpec((TM, D), lambda m: (m, 0)),
        pl.BlockSpec((TM, 1), lambda m: (m, 0)),
        pl.BlockSpec((TM, D), lambda m: (m, 0)),
    ],
    out_specs=[
        pl.BlockSpec((TM, D), lambda m: (m, 0)),
        pl.BlockSpec((1, 1), lambda m: (0, 0)),
        pl.BlockSpec((TM, D), lambda m: (m, 0)),
    ],
    out_shape=[
        jax.ShapeDtypeStruct((M, D), jnp.float32),
        jax.ShapeDtypeStruct((1, 1), jnp.float32),
        jax.ShapeDtypeStruct((M, D), jnp.float32),
    ],
)


@functools.cache
def _make_sc_gather():
    @functools.partial(
        pl.kernel,
        mesh=plsc.VectorSubcoreMesh(core_axis_name="c", subcore_axis_name="s"),
        out_type=jax.ShapeDtypeStruct((M, D), jnp.float32),
        scratch_types=[
            pltpu.VMEM((IDX_CH, IDX_CS), jnp.int32),
            pltpu.VMEM((BPW, D), jnp.float32),
            pltpu.SemaphoreType.DMA,
        ],
    )
    def _sc_gather(table_hbm, idx_hbm, out_hbm, idx_v, rows_v, sem):
        wid = lax.axis_index("s") * NC + lax.axis_index("c")
        pltpu.sync_copy(idx_hbm.at[wid], idx_v)
        for j in range(IDX_CH):
            pltpu.async_copy(table_hbm.at[idx_v.at[j]],
                             rows_v.at[pl.ds(j * IDX_CS, IDX_CS)], sem).wait()
        pltpu.sync_copy(rows_v, out_hbm.at[pl.ds(wid * BPW, BPW)])

    return _sc_gather


def _gather_rows(e_n, idx):
    return jnp.take(e_n, idx.reshape(-1), axis=0)


def kernel(x, codebooks):
    residual = x
    idx_list, scal_list, ss_list = [], [], []
    xq = None
    for i in range(NQ):
        e = codebooks[i]
        e_n = e / (jnp.linalg.norm(e, axis=-1, keepdims=True) + 1e-12)
        r_norm = jnp.linalg.norm(residual, axis=-1, keepdims=True)
        idx = _argmax(residual.reshape(M, D), r_norm.reshape(M, 1), e_n)
        quant = _gather_rows(e_n, idx)
        scal = jnp.sum(residual * quant.reshape(B, T, D), axis=-1,
                       keepdims=True).reshape(M, 1)
        if i < NQ - 1:
            rnew, ss = _update(residual.reshape(M, D), quant, scal)
        else:
            rnew, ss, xq = _update_final(residual.reshape(M, D), quant, scal,
                                         x.reshape(M, D))
        residual = rnew.reshape(B, T, D)
        idx_list.append(idx.reshape(B, T))
        scal_list.append(scal.reshape(B, T))
        ss_list.append(ss[0, 0])
    n_el = jnp.float32(M * D)
    mean_loss = jnp.mean(jnp.stack([1.25 * s / n_el for s in ss_list]))
    all_idx = jnp.stack(idx_list, axis=-1)
    all_scal = jnp.stack(scal_list, axis=-1)
    return (xq.reshape(B, T, D), mean_loss, all_idx, all_scal)


# speedup vs baseline: 1.4796x; 1.4796x over previous
import functools

import jax
import jax.numpy as jnp
from jax import lax
from jax.experimental import pallas as pl
from jax.experimental.pallas import tpu as pltpu
from jax.experimental.pallas import tpu_sc as plsc

B, T, D = 16, 576, 256
M = B * T
K = 8192
NQ = 4
TM = 2304
TK = 1024

NC, NS = 2, 16
NW = NC * NS
BPW = M // NW
IDX_CH, IDX_CS = 3, 96


def _argmax_body(r_ref, n_ref, e_ref, idx_ref, rmax, ridx):
    k = pl.program_id(0)
    m = pl.program_id(1)
    nk = pl.num_programs(0)
    rn = r_ref[...] / (n_ref[...] + 1e-12)
    s = lax.dot_general(rn, e_ref[...], (((1,), (1,)), ((), ())),
                        preferred_element_type=jnp.float32)
    lmax = jnp.max(s, axis=1, keepdims=True)
    iota = lax.broadcasted_iota(jnp.int32, s.shape, 1)
    lidx = jnp.min(jnp.where(s == lmax, iota, jnp.int32(2**30)),
                   axis=1, keepdims=True) + k * TK
    msl = pl.ds(m * TM, TM)

    @pl.when(k == 0)
    def _():
        rmax[msl] = lmax
        ridx[msl] = lidx

    @pl.when(k > 0)
    def _():
        better = lmax > rmax[msl]
        rmax[msl] = jnp.where(better, lmax, rmax[msl])
        ridx[msl] = jnp.where(better, lidx, ridx[msl])

    @pl.when(k == nk - 1)
    def _():
        idx_ref[...] = ridx[msl]


_argmax = pl.pallas_call(
    _argmax_body,
    grid=(K // TK, M // TM),
    in_specs=[
        pl.BlockSpec((TM, D), lambda k, m: (m, 0)),
        pl.BlockSpec((TM, 1), lambda k, m: (m, 0)),
        pl.BlockSpec((TK, D), lambda k, m: (k, 0)),
    ],
    out_specs=pl.BlockSpec((TM, 1), lambda k, m: (m, 0)),
    out_shape=jax.ShapeDtypeStruct((M, 1), jnp.int32),
    scratch_shapes=[
        pltpu.VMEM((M, 1), jnp.float32),
        pltpu.VMEM((M, 1), jnp.int32),
    ],
)


def _update_body(r_ref, q_ref, s_ref, rnew_ref, ss_ref):
    m = pl.program_id(0)
    rn = r_ref[...] - s_ref[...] * q_ref[...]
    rnew_ref[...] = rn

    @pl.when(m == 0)
    def _():
        ss_ref[...] = jnp.zeros((1, 1), jnp.float32)

    ss_ref[...] += jnp.sum(rn * rn).reshape(1, 1)


_update = pl.pallas_call(
    _update_body,
    grid=(M // TM,),
    in_specs=[
        pl.BlockSpec((TM, D), lambda m: (m, 0)),
        pl.BlockSpec((TM, D), lambda m: (m, 0)),
        pl.BlockSpec((TM, 1), lambda m: (m, 0)),
    ],
    out_specs=[
        pl.BlockSpec((TM, D), lambda m: (m, 0)),
        pl.BlockSpec((1, 1), lambda m: (0, 0)),
    ],
    out_shape=[
        jax.ShapeDtypeStruct((M, D), jnp.float32),
        jax.ShapeDtypeStruct((1, 1), jnp.float32),
    ],
)


def _update_final_body(r_ref, q_ref, s_ref, x_ref, rnew_ref, ss_ref, xq_ref):
    m = pl.program_id(0)
    rn = r_ref[...] - s_ref[...] * q_ref[...]
    rnew_ref[...] = rn
    xq_ref[...] = x_ref[...] - rn

    @pl.when(m == 0)
    def _():
        ss_ref[...] = jnp.zeros((1, 1), jnp.float32)

    ss_ref[...] += jnp.sum(rn * rn).reshape(1, 1)


_update_final = pl.pallas_call(
    _update_final_body,
    grid=(M // TM,),
    in_specs=[
        pl.BlockSpec((TM, D), lambda m: (m, 0)),
        pl.BlockSpec((TM, D), lambda m: (m, 0)),
        pl.BlockSpec((TM, 1), lambda m: (m, 0)),
        pl.BlockSpec((TM, D), lambda m: (m, 0)),
    ],
    out_specs=[
        pl.BlockSpec((TM, D), lambda m: (m, 0)),
        pl.BlockSpec((1, 1), lambda m: (0, 0)),
        pl.BlockSpec((TM, D), lambda m: (m, 0)),
    ],
    out_shape=[
        jax.ShapeDtypeStruct((M, D), jnp.float32),
        jax.ShapeDtypeStruct((1, 1), jnp.float32),
        jax.ShapeDtypeStruct((M, D), jnp.float32),
    ],
)


@functools.cache
def _make_sc_gather():
    @functools.partial(
        pl.kernel,
        mesh=plsc.VectorSubcoreMesh(core_axis_name="c", subcore_axis_name="s"),
        out_type=jax.ShapeDtypeStruct((M, D), jnp.float32),
        scratch_types=[
            pltpu.VMEM((IDX_CH, IDX_CS), jnp.int32),
            pltpu.VMEM((BPW, D), jnp.float32),
            pltpu.SemaphoreType.DMA,
        ],
    )
    def _sc_gather(table_hbm, idx_hbm, out_hbm, idx_v, rows_v, sem):
        wid = lax.axis_index("s") * NC + lax.axis_index("c")
        pltpu.sync_copy(idx_hbm.at[wid], idx_v)
        for j in range(IDX_CH):
            pltpu.async_copy(table_hbm.at[idx_v.at[j]],
                             rows_v.at[pl.ds(j * IDX_CS, IDX_CS)], sem).wait()
        pltpu.sync_copy(rows_v, out_hbm.at[pl.ds(wid * BPW, BPW)])

    return _sc_gather


def _gather_rows(e_n, idx):
    return _make_sc_gather()(e_n, idx.reshape(NW, IDX_CH, IDX_CS))


def kernel(x, codebooks):
    residual = x
    idx_list, scal_list, ss_list = [], [], []
    xq = None
    for i in range(NQ):
        e = codebooks[i]
        e_n = e / (jnp.linalg.norm(e, axis=-1, keepdims=True) + 1e-12)
        r_norm = jnp.linalg.norm(residual, axis=-1, keepdims=True)
        idx = _argmax(residual.reshape(M, D), r_norm.reshape(M, 1), e_n)
        quant = _gather_rows(e_n, idx)
        scal = jnp.sum(residual * quant.reshape(B, T, D), axis=-1,
                       keepdims=True).reshape(M, 1)
        if i < NQ - 1:
            rnew, ss = _update(residual.reshape(M, D), quant, scal)
        else:
            rnew, ss, xq = _update_final(residual.reshape(M, D), quant, scal,
                                         x.reshape(M, D))
        residual = rnew.reshape(B, T, D)
        idx_list.append(idx.reshape(B, T))
        scal_list.append(scal.reshape(B, T))
        ss_list.append(ss[0, 0])
    n_el = jnp.float32(M * D)
    mean_loss = jnp.mean(jnp.stack([1.25 * s / n_el for s in ss_list]))
    all_idx = jnp.stack(idx_list, axis=-1)
    all_scal = jnp.stack(scal_list, axis=-1)
    return (xq.reshape(B, T, D), mean_loss, all_idx, all_scal)


# speedup vs baseline: 1.5510x; 1.0483x over previous
import functools

import jax
import jax.numpy as jnp
from jax import lax
from jax.experimental import pallas as pl
from jax.experimental.pallas import tpu as pltpu
from jax.experimental.pallas import tpu_sc as plsc

B, T, D = 16, 576, 256
M = B * T
K = 8192
NQ = 4
TM = 4608
TK = 1024

NC, NS = 2, 16
NW = NC * NS
BPW = M // NW
IDX_CH, IDX_CS = 3, 96


def _argmax_body(r_ref, n_ref, e_ref, idx_ref, rmax, ridx):
    k = pl.program_id(0)
    m = pl.program_id(1)
    nk = pl.num_programs(0)
    rn = r_ref[...] / (n_ref[...] + 1e-12)
    s = lax.dot_general(rn, e_ref[...], (((1,), (1,)), ((), ())),
                        preferred_element_type=jnp.float32)
    lmax = jnp.max(s, axis=1, keepdims=True)
    iota = lax.broadcasted_iota(jnp.int32, s.shape, 1)
    lidx = jnp.min(jnp.where(s == lmax, iota, jnp.int32(2**30)),
                   axis=1, keepdims=True) + k * TK
    msl = pl.ds(m * TM, TM)

    @pl.when(k == 0)
    def _():
        rmax[msl] = lmax
        ridx[msl] = lidx

    @pl.when(k > 0)
    def _():
        better = lmax > rmax[msl]
        rmax[msl] = jnp.where(better, lmax, rmax[msl])
        ridx[msl] = jnp.where(better, lidx, ridx[msl])

    @pl.when(k == nk - 1)
    def _():
        idx_ref[...] = ridx[msl]


_argmax = pl.pallas_call(
    _argmax_body,
    grid=(K // TK, M // TM),
    in_specs=[
        pl.BlockSpec((TM, D), lambda k, m: (m, 0)),
        pl.BlockSpec((TM, 1), lambda k, m: (m, 0)),
        pl.BlockSpec((TK, D), lambda k, m: (k, 0)),
    ],
    out_specs=pl.BlockSpec((TM, 1), lambda k, m: (m, 0)),
    out_shape=jax.ShapeDtypeStruct((M, 1), jnp.int32),
    scratch_shapes=[
        pltpu.VMEM((M, 1), jnp.float32),
        pltpu.VMEM((M, 1), jnp.int32),
    ],
)


def _update_body(r_ref, q_ref, s_ref, rnew_ref, ss_ref):
    m = pl.program_id(0)
    rn = r_ref[...] - s_ref[...] * q_ref[...]
    rnew_ref[...] = rn

    @pl.when(m == 0)
    def _():
        ss_ref[...] = jnp.zeros((1, 1), jnp.float32)

    ss_ref[...] += jnp.sum(rn * rn).reshape(1, 1)


_update = pl.pallas_call(
    _update_body,
    grid=(M // TM,),
    in_specs=[
        pl.BlockSpec((TM, D), lambda m: (m, 0)),
        pl.BlockSpec((TM, D), lambda m: (m, 0)),
        pl.BlockSpec((TM, 1), lambda m: (m, 0)),
    ],
    out_specs=[
        pl.BlockSpec((TM, D), lambda m: (m, 0)),
        pl.BlockSpec((1, 1), lambda m: (0, 0)),
    ],
    out_shape=[
        jax.ShapeDtypeStruct((M, D), jnp.float32),
        jax.ShapeDtypeStruct((1, 1), jnp.float32),
    ],
)


def _update_final_body(r_ref, q_ref, s_ref, x_ref, rnew_ref, ss_ref, xq_ref):
    m = pl.program_id(0)
    rn = r_ref[...] - s_ref[...] * q_ref[...]
    rnew_ref[...] = rn
    xq_ref[...] = x_ref[...] - rn

    @pl.when(m == 0)
    def _():
        ss_ref[...] = jnp.zeros((1, 1), jnp.float32)

    ss_ref[...] += jnp.sum(rn * rn).reshape(1, 1)


_update_final = pl.pallas_call(
    _update_final_body,
    grid=(M // TM,),
    in_specs=[
        pl.BlockSpec((TM, D), lambda m: (m, 0)),
        pl.BlockSpec((TM, D), lambda m: (m, 0)),
        pl.BlockSpec((TM, 1), lambda m: (m, 0)),
        pl.BlockSpec((TM, D), lambda m: (m, 0)),
    ],
    out_specs=[
        pl.BlockSpec((TM, D), lambda m: (m, 0)),
        pl.BlockSpec((1, 1), lambda m: (0, 0)),
        pl.BlockSpec((TM, D), lambda m: (m, 0)),
    ],
    out_shape=[
        jax.ShapeDtypeStruct((M, D), jnp.float32),
        jax.ShapeDtypeStruct((1, 1), jnp.float32),
        jax.ShapeDtypeStruct((M, D), jnp.float32),
    ],
)


@functools.cache
def _make_sc_gather():
    @functools.partial(
        pl.kernel,
        mesh=plsc.VectorSubcoreMesh(core_axis_name="c", subcore_axis_name="s"),
        out_type=jax.ShapeDtypeStruct((M, D), jnp.float32),
        scratch_types=[
            pltpu.VMEM((IDX_CH, IDX_CS), jnp.int32),
            pltpu.VMEM((BPW, D), jnp.float32),
            pltpu.SemaphoreType.DMA,
        ],
    )
    def _sc_gather(table_hbm, idx_hbm, out_hbm, idx_v, rows_v, sem):
        wid = lax.axis_index("s") * NC + lax.axis_index("c")
        pltpu.sync_copy(idx_hbm.at[wid], idx_v)
        for j in range(IDX_CH):
            pltpu.async_copy(table_hbm.at[idx_v.at[j]],
                             rows_v.at[pl.ds(j * IDX_CS, IDX_CS)], sem).wait()
        pltpu.sync_copy(rows_v, out_hbm.at[pl.ds(wid * BPW, BPW)])

    return _sc_gather


def _gather_rows(e_n, idx):
    return _make_sc_gather()(e_n, idx.reshape(NW, IDX_CH, IDX_CS))


def kernel(x, codebooks):
    residual = x
    idx_list, scal_list, ss_list = [], [], []
    xq = None
    for i in range(NQ):
        e = codebooks[i]
        e_n = e / (jnp.linalg.norm(e, axis=-1, keepdims=True) + 1e-12)
        r_norm = jnp.linalg.norm(residual, axis=-1, keepdims=True)
        idx = _argmax(residual.reshape(M, D), r_norm.reshape(M, 1), e_n)
        quant = _gather_rows(e_n, idx)
        scal = jnp.sum(residual * quant.reshape(B, T, D), axis=-1,
                       keepdims=True).reshape(M, 1)
        if i < NQ - 1:
            rnew, ss = _update(residual.reshape(M, D), quant, scal)
        else:
            rnew, ss, xq = _update_final(residual.reshape(M, D), quant, scal,
                                         x.reshape(M, D))
        residual = rnew.reshape(B, T, D)
        idx_list.append(idx.reshape(B, T))
        scal_list.append(scal.reshape(B, T))
        ss_list.append(ss[0, 0])
    n_el = jnp.float32(M * D)
    mean_loss = jnp.mean(jnp.stack([1.25 * s / n_el for s in ss_list]))
    all_idx = jnp.stack(idx_list, axis=-1)
    all_scal = jnp.stack(scal_list, axis=-1)
    return (xq.reshape(B, T, D), mean_loss, all_idx, all_scal)


# speedup vs baseline: 1.5891x; 1.0245x over previous
import functools

import jax
import jax.numpy as jnp
from jax import lax
from jax.experimental import pallas as pl
from jax.experimental.pallas import tpu as pltpu
from jax.experimental.pallas import tpu_sc as plsc

B, T, D = 16, 576, 256
M = B * T
K = 8192
NQ = 4
TM = 4608
TK = 1024

NC, NS = 2, 16
NW = NC * NS
BPW = M // NW
IDX_CH, IDX_CS = 3, 96


def _argmax_body(r_ref, n_ref, e_ref, idx_ref, rmax, ridx):
    k = pl.program_id(0)
    m = pl.program_id(1)
    nk = pl.num_programs(0)
    rn = r_ref[...] / (n_ref[...] + 1e-12)
    s = lax.dot_general(e_ref[...], rn, (((1,), (1,)), ((), ())),
                        preferred_element_type=jnp.float32)
    lmax = jnp.max(s, axis=0, keepdims=True)
    iota = lax.broadcasted_iota(jnp.int32, s.shape, 0)
    lidx = jnp.min(jnp.where(s == lmax, iota, jnp.int32(2**30)),
                   axis=0, keepdims=True) + k * TK
    msl = (slice(None), pl.ds(m * TM, TM))

    @pl.when(k == 0)
    def _():
        rmax[msl] = lmax
        ridx[msl] = lidx

    @pl.when(k > 0)
    def _():
        better = lmax > rmax[msl]
        rmax[msl] = jnp.where(better, lmax, rmax[msl])
        ridx[msl] = jnp.where(better, lidx, ridx[msl])

    @pl.when(k == nk - 1)
    def _():
        idx_ref[...] = ridx[msl]


_argmax = pl.pallas_call(
    _argmax_body,
    grid=(K // TK, M // TM),
    in_specs=[
        pl.BlockSpec((TM, D), lambda k, m: (m, 0)),
        pl.BlockSpec((TM, 1), lambda k, m: (m, 0)),
        pl.BlockSpec((TK, D), lambda k, m: (k, 0)),
    ],
    out_specs=pl.BlockSpec((1, TM), lambda k, m: (0, m)),
    out_shape=jax.ShapeDtypeStruct((1, M), jnp.int32),
    scratch_shapes=[
        pltpu.VMEM((1, M), jnp.float32),
        pltpu.VMEM((1, M), jnp.int32),
    ],
)


def _update_body(r_ref, q_ref, s_ref, rnew_ref, ss_ref):
    m = pl.program_id(0)
    rn = r_ref[...] - s_ref[...] * q_ref[...]
    rnew_ref[...] = rn

    @pl.when(m == 0)
    def _():
        ss_ref[...] = jnp.zeros((1, 1), jnp.float32)

    ss_ref[...] += jnp.sum(rn * rn).reshape(1, 1)


_update = pl.pallas_call(
    _update_body,
    grid=(M // TM,),
    in_specs=[
        pl.BlockSpec((TM, D), lambda m: (m, 0)),
        pl.BlockSpec((TM, D), lambda m: (m, 0)),
        pl.BlockSpec((TM, 1), lambda m: (m, 0)),
    ],
    out_specs=[
        pl.BlockSpec((TM, D), lambda m: (m, 0)),
        pl.BlockSpec((1, 1), lambda m: (0, 0)),
    ],
    out_shape=[
        jax.ShapeDtypeStruct((M, D), jnp.float32),
        jax.ShapeDtypeStruct((1, 1), jnp.float32),
    ],
)


def _update_final_body(r_ref, q_ref, s_ref, x_ref, rnew_ref, ss_ref, xq_ref):
    m = pl.program_id(0)
    rn = r_ref[...] - s_ref[...] * q_ref[...]
    rnew_ref[...] = rn
    xq_ref[...] = x_ref[...] - rn

    @pl.when(m == 0)
    def _():
        ss_ref[...] = jnp.zeros((1, 1), jnp.float32)

    ss_ref[...] += jnp.sum(rn * rn).reshape(1, 1)


_update_final = pl.pallas_call(
    _update_final_body,
    grid=(M // TM,),
    in_specs=[
        pl.BlockSpec((TM, D), lambda m: (m, 0)),
        pl.BlockSpec((TM, D), lambda m: (m, 0)),
        pl.BlockSpec((TM, 1), lambda m: (m, 0)),
        pl.BlockSpec((TM, D), lambda m: (m, 0)),
    ],
    out_specs=[
        pl.BlockSpec((TM, D), lambda m: (m, 0)),
        pl.BlockSpec((1, 1), lambda m: (0, 0)),
        pl.BlockSpec((TM, D), lambda m: (m, 0)),
    ],
    out_shape=[
        jax.ShapeDtypeStruct((M, D), jnp.float32),
        jax.ShapeDtypeStruct((1, 1), jnp.float32),
        jax.ShapeDtypeStruct((M, D), jnp.float32),
    ],
)


@functools.cache
def _make_sc_gather():
    @functools.partial(
        pl.kernel,
        mesh=plsc.VectorSubcoreMesh(core_axis_name="c", subcore_axis_name="s"),
        out_type=jax.ShapeDtypeStruct((M, D), jnp.float32),
        scratch_types=[
            pltpu.VMEM((IDX_CH, IDX_CS), jnp.int32),
            pltpu.VMEM((BPW, D), jnp.float32),
            pltpu.SemaphoreType.DMA,
        ],
    )
    def _sc_gather(table_hbm, idx_hbm, out_hbm, idx_v, rows_v, sem):
        wid = lax.axis_index("s") * NC + lax.axis_index("c")
        pltpu.sync_copy(idx_hbm.at[wid], idx_v)
        for j in range(IDX_CH):
            pltpu.async_copy(table_hbm.at[idx_v.at[j]],
                             rows_v.at[pl.ds(j * IDX_CS, IDX_CS)], sem).wait()
        pltpu.sync_copy(rows_v, out_hbm.at[pl.ds(wid * BPW, BPW)])

    return _sc_gather


def _gather_rows(e_n, idx):
    return _make_sc_gather()(e_n, idx.reshape(NW, IDX_CH, IDX_CS))


def kernel(x, codebooks):
    residual = x
    idx_list, scal_list, ss_list = [], [], []
    xq = None
    for i in range(NQ):
        e = codebooks[i]
        e_n = e / (jnp.linalg.norm(e, axis=-1, keepdims=True) + 1e-12)
        r_norm = jnp.linalg.norm(residual, axis=-1, keepdims=True)
        idx = _argmax(residual.reshape(M, D), r_norm.reshape(M, 1), e_n)
        quant = _gather_rows(e_n, idx)
        scal = jnp.sum(residual * quant.reshape(B, T, D), axis=-1,
                       keepdims=True).reshape(M, 1)
        if i < NQ - 1:
            rnew, ss = _update(residual.reshape(M, D), quant, scal)
        else:
            rnew, ss, xq = _update_final(residual.reshape(M, D), quant, scal,
                                         x.reshape(M, D))
        residual = rnew.reshape(B, T, D)
        idx_list.append(idx.reshape(B, T))
        scal_list.append(scal.reshape(B, T))
        ss_list.append(ss[0, 0])
    n_el = jnp.float32(M * D)
    mean_loss = jnp.mean(jnp.stack([1.25 * s / n_el for s in ss_list]))
    all_idx = jnp.stack(idx_list, axis=-1)
    all_scal = jnp.stack(scal_list, axis=-1)
    return (xq.reshape(B, T, D), mean_loss, all_idx, all_scal)
